# baseline (device time: 426841 ns/iter reference)
import jax
import jax.numpy as jnp
from jax import lax
from jax.experimental import pallas as pl
from jax.experimental.pallas import tpu as pltpu

M = 16384
N_OUT = 1024
H = M // 2
CH = 256
K = H // CH
NSLOT = 4


def kernel(x):
    assert x.shape == (1, M, 2 * N_OUT), x.shape

    def body(x_ref, out_ref, send_buf, recv_x, local_buf, red_buf, recv_y,
             x_send_sems, x_recv_sems, y_send_sems, y_recv_sems,
             in_sems, src_sems, out1_sems, out2_sems):
        my_x = lax.axis_index("x")
        my_y = lax.axis_index("y")
        peer_x = 1 - my_x
        peer_y = 1 - my_y
        my_col0 = my_x * N_OUT
        peer_col0 = peer_x * N_OUT
        half0 = my_y * H
        ohalf0 = peer_y * H

        def make_src(c):
            s = c % NSLOT
            return pltpu.make_async_copy(
                x_ref.at[0, pl.ds(half0 + c * CH, CH), pl.ds(peer_col0, N_OUT)],
                send_buf.at[s],
                src_sems.at[s],
            )

        def make_x(c):
            s = c % NSLOT
            return pltpu.make_async_remote_copy(
                src_ref=send_buf.at[s],
                dst_ref=recv_x.at[s],
                send_sem=x_send_sems.at[s],
                recv_sem=x_recv_sems.at[s],
                device_id=(peer_x, my_y),
                device_id_type=pl.DeviceIdType.MESH,
            )

        def make_in(c):
            s = c % NSLOT
            return pltpu.make_async_copy(
                x_ref.at[0, pl.ds(half0 + c * CH, CH), pl.ds(my_col0, N_OUT)],
                local_buf.at[s],
                in_sems.at[s],
            )

        def make_y(c):
            s = c % NSLOT
            return pltpu.make_async_remote_copy(
                src_ref=red_buf.at[s],
                dst_ref=recv_y.at[s],
                send_sem=y_send_sems.at[s],
                recv_sem=y_recv_sems.at[s],
                device_id=(my_x, peer_y),
                device_id_type=pl.DeviceIdType.MESH,
            )

        def make_out1(c):
            s = c % NSLOT
            return pltpu.make_async_copy(
                red_buf.at[s],
                out_ref.at[pl.ds(half0 + c * CH, CH), :],
                out1_sems.at[s % 2],
            )

        def make_out2(c):
            s = c % NSLOT
            return pltpu.make_async_copy(
                recv_y.at[s],
                out_ref.at[pl.ds(ohalf0 + c * CH, CH), :],
                out2_sems.at[s % 2],
            )

        make_src(0).start()
        make_src(1).start()
        make_in(0).start()
        make_src(0).wait()
        make_x(0).start()

        for c in range(K):
            s = c % NSLOT
            if c + 2 < K:
                make_src(c + 2).start()
            if c + 1 < K:
                make_in(c + 1).start()
                make_src(c + 1).wait()
                make_x(c + 1).start()
            make_in(c).wait()
            make_x(c).wait()
            if c >= 1:
                make_out1(c - 1).wait()
            if c >= 2:
                make_out2(c - 2).wait()

            red_buf[s] = local_buf[s] + recv_x[s]

            make_out1(c).start()
            make_y(c).start()
            if c >= 1:
                make_y(c - 1).wait()
                make_out2(c - 1).start()

        make_y(K - 1).wait()
        make_out2(K - 1).start()
        make_out1(K - 1).wait()
        make_out2(K - 2).wait()
        make_out2(K - 1).wait()

    return pl.pallas_call(
        body,
        out_shape=jax.ShapeDtypeStruct((M, N_OUT), jnp.float32),
        in_specs=[pl.BlockSpec(memory_space=pl.ANY)],
        out_specs=pl.BlockSpec(memory_space=pl.ANY),
        scratch_shapes=[
            pltpu.VMEM((NSLOT, CH, N_OUT), jnp.float32),
            pltpu.VMEM((NSLOT, CH, N_OUT), jnp.float32),
            pltpu.VMEM((NSLOT, CH, N_OUT), jnp.float32),
            pltpu.VMEM((NSLOT, CH, N_OUT), jnp.float32),
            pltpu.VMEM((NSLOT, CH, N_OUT), jnp.float32),
            pltpu.SemaphoreType.DMA((NSLOT,)),
            pltpu.SemaphoreType.DMA((NSLOT,)),
            pltpu.SemaphoreType.DMA((NSLOT,)),
            pltpu.SemaphoreType.DMA((NSLOT,)),
            pltpu.SemaphoreType.DMA((NSLOT,)),
            pltpu.SemaphoreType.DMA((NSLOT,)),
            pltpu.SemaphoreType.DMA((2,)),
            pltpu.SemaphoreType.DMA((2,)),
        ],
        compiler_params=pltpu.CompilerParams(
            vmem_limit_bytes=100 * 1024 * 1024,
        ),
    )(x)


# device time: 423451 ns/iter; 1.0080x vs baseline; 1.0080x over previous
import jax
import jax.numpy as jnp
from jax import lax
from jax.experimental import pallas as pl
from jax.experimental.pallas import tpu as pltpu

M = 16384
N_OUT = 1024
H = M // 2
CH = 256
K = H // CH
NSLOT = 4


def kernel(x):
    assert x.shape == (1, M, 2 * N_OUT), x.shape

    def body(x_ref, out_ref, send_buf, recv_x, local_buf, red_buf, recv_y,
             x_send_sems, x_recv_sems, y_send_sems, y_recv_sems,
             in_sems, src_sems, out1_sems, out2_sems):
        my_x = lax.axis_index("x")
        my_y = lax.axis_index("y")
        peer_x = 1 - my_x
        peer_y = 1 - my_y
        my_col0 = my_x * N_OUT
        peer_col0 = peer_x * N_OUT
        half0 = my_y * H
        ohalf0 = peer_y * H

        def make_src(c):
            s = c % NSLOT
            return pltpu.make_async_copy(
                x_ref.at[0, pl.ds(half0 + c * CH, CH), pl.ds(peer_col0, N_OUT)],
                send_buf.at[s],
                src_sems.at[s],
            )

        def make_x(c):
            s = c % NSLOT
            return pltpu.make_async_remote_copy(
                src_ref=send_buf.at[s],
                dst_ref=recv_x.at[s],
                send_sem=x_send_sems.at[s],
                recv_sem=x_recv_sems.at[s],
                device_id=(peer_x, my_y),
                device_id_type=pl.DeviceIdType.MESH,
            )

        def make_in(c):
            s = c % NSLOT
            return pltpu.make_async_copy(
                x_ref.at[0, pl.ds(half0 + c * CH, CH), pl.ds(my_col0, N_OUT)],
                local_buf.at[s],
                in_sems.at[s],
            )

        def make_y(c):
            s = c % NSLOT
            return pltpu.make_async_remote_copy(
                src_ref=red_buf.at[s],
                dst_ref=recv_y.at[s],
                send_sem=y_send_sems.at[s],
                recv_sem=y_recv_sems.at[s],
                device_id=(my_x, peer_y),
                device_id_type=pl.DeviceIdType.MESH,
            )

        def make_out1(c):
            s = c % NSLOT
            return pltpu.make_async_copy(
                red_buf.at[s],
                out_ref.at[pl.ds(half0 + c * CH, CH), :],
                out1_sems.at[s % 2],
            )

        def make_out2(c):
            s = c % NSLOT
            return pltpu.make_async_copy(
                recv_y.at[s],
                out_ref.at[pl.ds(ohalf0 + c * CH, CH), :],
                out2_sems.at[s % 2],
            )

        barrier_sem = pltpu.get_barrier_semaphore()
        pl.semaphore_signal(barrier_sem, inc=1, device_id=(peer_x, my_y),
                            device_id_type=pl.DeviceIdType.MESH)
        pl.semaphore_signal(barrier_sem, inc=1, device_id=(my_x, peer_y),
                            device_id_type=pl.DeviceIdType.MESH)
        pl.semaphore_wait(barrier_sem, 2)

        make_src(0).start()
        make_src(1).start()
        make_in(0).start()
        make_src(0).wait()
        make_x(0).start()

        for c in range(K):
            s = c % NSLOT
            if c + 2 < K:
                make_src(c + 2).start()
            if c + 1 < K:
                make_in(c + 1).start()
                make_src(c + 1).wait()
                make_x(c + 1).start()
            make_in(c).wait()
            make_x(c).wait()
            if c >= 1:
                make_out1(c - 1).wait()
            if c >= 2:
                make_out2(c - 2).wait()

            red_buf[s] = local_buf[s] + recv_x[s]

            make_out1(c).start()
            make_y(c).start()
            if c >= 1:
                make_y(c - 1).wait()
                make_out2(c - 1).start()

        make_y(K - 1).wait()
        make_out2(K - 1).start()
        make_out1(K - 1).wait()
        make_out2(K - 2).wait()
        make_out2(K - 1).wait()

    return pl.pallas_call(
        body,
        out_shape=jax.ShapeDtypeStruct((M, N_OUT), jnp.float32),
        in_specs=[pl.BlockSpec(memory_space=pl.ANY)],
        out_specs=pl.BlockSpec(memory_space=pl.ANY),
        scratch_shapes=[
            pltpu.VMEM((NSLOT, CH, N_OUT), jnp.float32),
            pltpu.VMEM((NSLOT, CH, N_OUT), jnp.float32),
            pltpu.VMEM((NSLOT, CH, N_OUT), jnp.float32),
            pltpu.VMEM((NSLOT, CH, N_OUT), jnp.float32),
            pltpu.VMEM((NSLOT, CH, N_OUT), jnp.float32),
            pltpu.SemaphoreType.DMA((NSLOT,)),
            pltpu.SemaphoreType.DMA((NSLOT,)),
            pltpu.SemaphoreType.DMA((NSLOT,)),
            pltpu.SemaphoreType.DMA((NSLOT,)),
            pltpu.SemaphoreType.DMA((NSLOT,)),
            pltpu.SemaphoreType.DMA((NSLOT,)),
            pltpu.SemaphoreType.DMA((2,)),
            pltpu.SemaphoreType.DMA((2,)),
        ],
        compiler_params=pltpu.CompilerParams(
            vmem_limit_bytes=100 * 1024 * 1024,
            collective_id=0,
        ),
    )(x)


# device time: 417446 ns/iter; 1.0225x vs baseline; 1.0144x over previous
import jax
import jax.numpy as jnp
from jax import lax
from jax.experimental import pallas as pl
from jax.experimental.pallas import tpu as pltpu

M = 16384
N_OUT = 1024
H = M // 2
CH = 128
K = H // CH
NSLOT = 4


def kernel(x):
    assert x.shape == (1, M, 2 * N_OUT), x.shape

    def body(x_ref, out_ref, send_buf, recv_x, local_buf, red_buf, recv_y,
             x_send_sems, x_recv_sems, y_send_sems, y_recv_sems,
             in_sems, src_sems, out1_sems, out2_sems):
        my_x = lax.axis_index("x")
        my_y = lax.axis_index("y")
        peer_x = 1 - my_x
        peer_y = 1 - my_y
        my_col0 = my_x * N_OUT
        peer_col0 = peer_x * N_OUT
        half0 = my_y * H
        ohalf0 = peer_y * H

        def make_src(c):
            s = c % NSLOT
            return pltpu.make_async_copy(
                x_ref.at[0, pl.ds(half0 + c * CH, CH), pl.ds(peer_col0, N_OUT)],
                send_buf.at[s],
                src_sems.at[s],
            )

        def make_x(c):
            s = c % NSLOT
            return pltpu.make_async_remote_copy(
                src_ref=send_buf.at[s],
                dst_ref=recv_x.at[s],
                send_sem=x_send_sems.at[s],
                recv_sem=x_recv_sems.at[s],
                device_id=(peer_x, my_y),
                device_id_type=pl.DeviceIdType.MESH,
            )

        def make_in(c):
            s = c % NSLOT
            return pltpu.make_async_copy(
                x_ref.at[0, pl.ds(half0 + c * CH, CH), pl.ds(my_col0, N_OUT)],
                local_buf.at[s],
                in_sems.at[s],
            )

        def make_y(c):
            s = c % NSLOT
            return pltpu.make_async_remote_copy(
                src_ref=red_buf.at[s],
                dst_ref=recv_y.at[s],
                send_sem=y_send_sems.at[s],
                recv_sem=y_recv_sems.at[s],
                device_id=(my_x, peer_y),
                device_id_type=pl.DeviceIdType.MESH,
            )

        def make_out1(c):
            s = c % NSLOT
            return pltpu.make_async_copy(
                red_buf.at[s],
                out_ref.at[pl.ds(half0 + c * CH, CH), :],
                out1_sems.at[s % 2],
            )

        def make_out2(c):
            s = c % NSLOT
            return pltpu.make_async_copy(
                recv_y.at[s],
                out_ref.at[pl.ds(ohalf0 + c * CH, CH), :],
                out2_sems.at[s % 2],
            )

        barrier_sem = pltpu.get_barrier_semaphore()
        pl.semaphore_signal(barrier_sem, inc=1, device_id=(peer_x, my_y),
                            device_id_type=pl.DeviceIdType.MESH)
        pl.semaphore_signal(barrier_sem, inc=1, device_id=(my_x, peer_y),
                            device_id_type=pl.DeviceIdType.MESH)
        pl.semaphore_wait(barrier_sem, 2)

        make_src(0).start()
        make_src(1).start()
        make_in(0).start()
        make_src(0).wait()
        make_x(0).start()

        for c in range(K):
            s = c % NSLOT
            if c + 2 < K:
                make_src(c + 2).start()
            if c + 1 < K:
                make_in(c + 1).start()
                make_src(c + 1).wait()
                make_x(c + 1).start()
            make_in(c).wait()
            make_x(c).wait()
            if c >= 1:
                make_out1(c - 1).wait()
            if c >= 2:
                make_out2(c - 2).wait()

            red_buf[s] = local_buf[s] + recv_x[s]

            make_out1(c).start()
            make_y(c).start()
            if c >= 1:
                make_y(c - 1).wait()
                make_out2(c - 1).start()

        make_y(K - 1).wait()
        make_out2(K - 1).start()
        make_out1(K - 1).wait()
        make_out2(K - 2).wait()
        make_out2(K - 1).wait()

    return pl.pallas_call(
        body,
        out_shape=jax.ShapeDtypeStruct((M, N_OUT), jnp.float32),
        in_specs=[pl.BlockSpec(memory_space=pl.ANY)],
        out_specs=pl.BlockSpec(memory_space=pl.ANY),
        scratch_shapes=[
            pltpu.VMEM((NSLOT, CH, N_OUT), jnp.float32),
            pltpu.VMEM((NSLOT, CH, N_OUT), jnp.float32),
            pltpu.VMEM((NSLOT, CH, N_OUT), jnp.float32),
            pltpu.VMEM((NSLOT, CH, N_OUT), jnp.float32),
            pltpu.VMEM((NSLOT, CH, N_OUT), jnp.float32),
            pltpu.SemaphoreType.DMA((NSLOT,)),
            pltpu.SemaphoreType.DMA((NSLOT,)),
            pltpu.SemaphoreType.DMA((NSLOT,)),
            pltpu.SemaphoreType.DMA((NSLOT,)),
            pltpu.SemaphoreType.DMA((NSLOT,)),
            pltpu.SemaphoreType.DMA((NSLOT,)),
            pltpu.SemaphoreType.DMA((2,)),
            pltpu.SemaphoreType.DMA((2,)),
        ],
        compiler_params=pltpu.CompilerParams(
            vmem_limit_bytes=100 * 1024 * 1024,
            collective_id=0,
        ),
    )(x)
